# baseline (speedup 1.0000x reference)
import functools

import jax
import jax.numpy as jnp
from jax import lax
from jax.experimental import pallas as pl
from jax.experimental.pallas import tpu as pltpu
from jax.experimental.pallas import tpu_sc as plsc

VOCAB = 100000
EMB = 128
BATCH = 4096
SEQ = 200
OUT = 19
OUT_PAD = 32
_PACK = 8
_QUARTER = VOCAB // 4

_TC_BLK = 5000
_PREC = lax.Precision.HIGHEST


def _tc_fold_body(t0_ref, t1_ref, t2q_ref, t3_ref, w1_ref, w2_ref,
                  b1_ref, b2_ref, t4_ref, bias_ref):
    w2p = jnp.pad(w2_ref[...], ((0, 0), (0, OUT_PAD - OUT)))
    b2p = jnp.pad(b2_ref[...], ((0, 0), (0, OUT_PAD - OUT)))
    w12 = jnp.dot(w1_ref[...], w2p, preferred_element_type=jnp.float32,
                  precision=_PREC).astype(jnp.bfloat16)
    wt = jnp.tile(w12, (4, 4))
    rowa = lax.broadcasted_iota(jnp.int32, (4 * EMB, EMB), 0) // EMB
    cola = lax.broadcasted_iota(jnp.int32, (4 * EMB, EMB), 1) // OUT_PAD
    w_bd = jnp.where(rowa == cola, wt, jnp.bfloat16(0))
    cat = jnp.concatenate(
        [t0_ref[...], t1_ref[...], t2q_ref[...], t3_ref[...]],
        axis=1).astype(jnp.bfloat16)
    hc = jnp.dot(cat, w_bd, preferred_element_type=jnp.float32)
    u = lax.bitcast_convert_type(hc, jnp.uint32)
    r = u + 0x7FFF + ((u >> 16) & 1)
    hi = r & jnp.uint32(0xFFFF0000)
    packed = hi | (jnp.roll(hi, -16, axis=1) >> 16)
    t4_ref[...] = lax.bitcast_convert_type(packed, jnp.float32)
    bias_ref[...] = jnp.dot(b1_ref[...], w2p,
                            preferred_element_type=jnp.float32,
                            precision=_PREC) + b2p


def _quarter_spec(a):
    return pl.BlockSpec((_TC_BLK, EMB),
                        lambda i, a=a: (a * (_QUARTER // _TC_BLK) + i, 0))


def _fold_table(table, W1, W2, b1r, b2r):
    return pl.pallas_call(
        _tc_fold_body,
        grid=(_QUARTER // _TC_BLK,),
        in_specs=[
            _quarter_spec(0),
            _quarter_spec(1),
            _quarter_spec(2),
            _quarter_spec(3),
            pl.BlockSpec((EMB, EMB), lambda i: (0, 0)),
            pl.BlockSpec((EMB, OUT), lambda i: (0, 0)),
            pl.BlockSpec((1, EMB), lambda i: (0, 0)),
            pl.BlockSpec((1, OUT), lambda i: (0, 0)),
        ],
        out_specs=[
            pl.BlockSpec((_TC_BLK, EMB), lambda i: (i, 0)),
            pl.BlockSpec((1, OUT_PAD), lambda i: (0, 0)),
        ],
        out_shape=[
            jax.ShapeDtypeStruct((_QUARTER, EMB), jnp.float32),
            jax.ShapeDtypeStruct((1, OUT_PAD), jnp.float32),
        ],
    )(table, table, table, table, W1, W2, b1r, b2r)


_INFO = plsc.get_sparse_core_info()
_NC, _NS, _L = _INFO.num_cores, _INFO.num_subcores, _INFO.num_lanes
_NW = _NC * _NS
_B_PER_W = BATCH // _NW
_SPLIT0 = 104
_SPLIT1 = SEQ - _SPLIT0
_CHUNK_ROWS = 8
_N_CHUNKS = _B_PER_W // _CHUNK_ROWS
_BUF_ROWS = _CHUNK_ROWS * SEQ
_VSLOTS = [k * _L for k in range(SEQ // _L)] + [SEQ - _L]
_HIMASK = 0xFFFF0000


def _sc_pool_body(x_hbm, t2_hbm, bias_hbm, out_hbm,
                  idx_v, buf_a, buf_b, out_v, bias_v, sem_a, sem_b):
    wid = lax.axis_index("s") * _NC + lax.axis_index("c")
    pltpu.sync_copy(x_hbm.at[pl.ds(wid * _B_PER_W, _B_PER_W)], idx_v)
    pltpu.sync_copy(bias_hbm, bias_v)
    bias0 = bias_v[pl.ds(0, _L)]
    bias1 = bias_v[pl.ds(_L, _L)]

    def remap_row(row):
        vs = [idx_v[row, pl.ds(o, _L)] for o in _VSLOTS]
        for o, v in zip(_VSLOTS, vs):
            q = jax.lax.shift_right_logical(v * 21475, 29)
            idx_v[row, pl.ds(o, _L)] = v * _PACK - q * (_PACK * _QUARTER - 2)

    def fire(c, buf, sem):
        for r in range(_CHUNK_ROWS):
            row = _CHUNK_ROWS * c + r
            remap_row(row)
            pltpu.async_copy(
                t2_hbm.at[idx_v.at[row, pl.ds(0, _SPLIT0)]],
                buf.at[pl.ds(r * SEQ, _SPLIT0)], sem)
            pltpu.async_copy(
                t2_hbm.at[idx_v.at[row, pl.ds(_SPLIT0, _SPLIT1)]],
                buf.at[pl.ds(r * SEQ + _SPLIT0, _SPLIT1)], sem)

    def drain(buf, sem):
        pltpu.make_async_copy(
            t2_hbm.at[pl.ds(0, _BUF_ROWS)], buf, sem).wait()

    def group8(o, buf):
        rows = [plsc.bitcast(buf[o + jj, pl.ds(0, _L)],
                             jnp.bfloat16) for jj in range(8)]
        p01, p23 = rows[0] + rows[1], rows[2] + rows[3]
        p45, p67 = rows[4] + rows[5], rows[6] + rows[7]
        g = (p01 + p23) + (p45 + p67)
        w = plsc.bitcast(g, jnp.uint32)
        f_hi = plsc.bitcast(w & jnp.uint32(_HIMASK), jnp.float32)
        f_lo = plsc.bitcast(w << 16, jnp.float32)
        return f_hi, f_lo

    def accum_chunk(c, buf):
        for r in range(_CHUNK_ROWS):
            base = r * SEQ

            def jbody(ji, accs, base=base):
                o = base + ji * 16
                h0, l0 = group8(o, buf)
                h1, l1 = group8(o + 8, buf)
                return (accs[0] + h0, accs[1] + l0,
                        accs[2] + h1, accs[3] + l1)

            z = jnp.zeros((_L,), jnp.float32)
            accs = lax.fori_loop(0, SEQ // 16, jbody, (z, z, z, z))
            hT, lT = group8(base + SEQ - 8, buf)
            a0 = (accs[0] + accs[2]) + hT
            a1 = (accs[1] + accs[3]) + lT
            rg = c * _CHUNK_ROWS + r
            out_v[rg, pl.ds(0, _L)] = a0 + bias0
            out_v[rg, pl.ds(_L, _L)] = a1 + bias1

    fire(0, buf_a, sem_a)

    def body(i, carry):
        for b in range(2):
            cbuf, csem = (buf_a, sem_a) if b == 0 else (buf_b, sem_b)
            nbuf, nsem = (buf_b, sem_b) if b == 0 else (buf_a, sem_a)
            c = 2 * i + b

            @pl.when(c < _N_CHUNKS - 1)
            def _(c=c, nbuf=nbuf, nsem=nsem):
                fire(c + 1, nbuf, nsem)

            drain(cbuf, csem)
            accum_chunk(c, cbuf)
        return carry

    lax.fori_loop(0, _N_CHUNKS // 2, body, 0)
    pltpu.sync_copy(out_v, out_hbm.at[pl.ds(wid * _B_PER_W, _B_PER_W)])


_sc_pool = functools.partial(
    pl.kernel,
    out_type=jax.ShapeDtypeStruct((BATCH, OUT_PAD), jnp.float32),
    mesh=plsc.VectorSubcoreMesh(core_axis_name="c", subcore_axis_name="s"),
    compiler_params=pltpu.CompilerParams(use_tc_tiling_on_sc=False,
                                         needs_layout_passes=False),
    scratch_types=[
        pltpu.VMEM((_B_PER_W, SEQ), jnp.int32),
        pltpu.VMEM((_BUF_ROWS, _L), jnp.float32),
        pltpu.VMEM((_BUF_ROWS, _L), jnp.float32),
        pltpu.VMEM((_B_PER_W, OUT_PAD), jnp.float32),
        pltpu.VMEM((OUT_PAD,), jnp.float32),
        pltpu.SemaphoreType.DMA,
        pltpu.SemaphoreType.DMA,
    ],
)(_sc_pool_body)


def kernel(x, table, W1, b1, W2, b2):
    b1r = b1.reshape(1, EMB)
    b2r = b2.reshape(1, OUT)
    t4, bias = _fold_table(table, W1, W2, b1r, b2r)
    t2 = t4.reshape(_PACK * _QUARTER, _L)
    pooled = _sc_pool(x.astype(jnp.int32), t2, bias.reshape(OUT_PAD))
    return pooled[:, :OUT]


# speedup vs baseline: 1.0182x; 1.0182x over previous
import functools

import jax
import jax.numpy as jnp
from jax import lax
from jax.experimental import pallas as pl
from jax.experimental.pallas import tpu as pltpu
from jax.experimental.pallas import tpu_sc as plsc

VOCAB = 100000
EMB = 128
BATCH = 4096
SEQ = 200
OUT = 19
OUT_PAD = 32
_PACK = 8
_QUARTER = VOCAB // 4

_TC_BLK = 5000
_PREC = lax.Precision.HIGHEST


def _tc_fold_body(t0_ref, t1_ref, t2q_ref, t3_ref, w1_ref, w2_ref,
                  b1_ref, b2_ref, t4_ref, bias_ref):
    w2p = jnp.pad(w2_ref[...], ((0, 0), (0, OUT_PAD - OUT)))
    b2p = jnp.pad(b2_ref[...], ((0, 0), (0, OUT_PAD - OUT)))
    w12 = jnp.dot(w1_ref[...], w2p, preferred_element_type=jnp.float32,
                  precision=_PREC).astype(jnp.bfloat16)
    wt = jnp.tile(w12, (4, 4))
    rowa = lax.broadcasted_iota(jnp.int32, (4 * EMB, EMB), 0) // EMB
    cola = lax.broadcasted_iota(jnp.int32, (4 * EMB, EMB), 1) // OUT_PAD
    w_bd = jnp.where(rowa == cola, wt, jnp.bfloat16(0))
    cat = jnp.concatenate(
        [t0_ref[...], t1_ref[...], t2q_ref[...], t3_ref[...]],
        axis=1).astype(jnp.bfloat16)
    hc = jnp.dot(cat, w_bd, preferred_element_type=jnp.float32)
    u = lax.bitcast_convert_type(hc, jnp.uint32)
    r = u + 0x7FFF + ((u >> 16) & 1)
    hi = r & jnp.uint32(0xFFFF0000)
    packed = hi | (jnp.roll(hi, -16, axis=1) >> 16)
    t4_ref[...] = lax.bitcast_convert_type(packed, jnp.float32)
    bias_ref[...] = jnp.dot(b1_ref[...], w2p,
                            preferred_element_type=jnp.float32,
                            precision=_PREC) + b2p


def _quarter_spec(a):
    return pl.BlockSpec((_TC_BLK, EMB),
                        lambda i, a=a: (a * (_QUARTER // _TC_BLK) + i, 0))


def _fold_table(table, W1, W2, b1r, b2r):
    return pl.pallas_call(
        _tc_fold_body,
        grid=(_QUARTER // _TC_BLK,),
        in_specs=[
            _quarter_spec(0),
            _quarter_spec(1),
            _quarter_spec(2),
            _quarter_spec(3),
            pl.BlockSpec((EMB, EMB), lambda i: (0, 0)),
            pl.BlockSpec((EMB, OUT), lambda i: (0, 0)),
            pl.BlockSpec((1, EMB), lambda i: (0, 0)),
            pl.BlockSpec((1, OUT), lambda i: (0, 0)),
        ],
        out_specs=[
            pl.BlockSpec((_TC_BLK, EMB), lambda i: (i, 0)),
            pl.BlockSpec((1, OUT_PAD), lambda i: (0, 0)),
        ],
        out_shape=[
            jax.ShapeDtypeStruct((_QUARTER, EMB), jnp.float32),
            jax.ShapeDtypeStruct((1, OUT_PAD), jnp.float32),
        ],
    )(table, table, table, table, W1, W2, b1r, b2r)


_INFO = plsc.get_sparse_core_info()
_NC, _NS, _L = _INFO.num_cores, _INFO.num_subcores, _INFO.num_lanes
_NW = _NC * _NS
_B_PER_W = BATCH // _NW
_SPLIT0 = 104
_SPLIT1 = SEQ - _SPLIT0
_CHUNK_ROWS = 8
_N_CHUNKS = _B_PER_W // _CHUNK_ROWS
_BUF_ROWS = _CHUNK_ROWS * SEQ
_VSLOTS = [k * _L for k in range(SEQ // _L)] + [SEQ - _L]
_HIMASK = 0xFFFF0000


def _sc_pool_body(x_hbm, t2_hbm, bias_hbm, out_hbm,
                  idx_v, buf_a, buf_b, out_v, bias_v, sem_a, sem_b):
    wid = lax.axis_index("s") * _NC + lax.axis_index("c")
    pltpu.sync_copy(x_hbm.at[pl.ds(wid * _B_PER_W, _B_PER_W)], idx_v)
    pltpu.sync_copy(bias_hbm, bias_v)
    bias0 = bias_v[pl.ds(0, _L)]
    bias1 = bias_v[pl.ds(_L, _L)]

    def remap_row(row):
        vs = [idx_v[row, pl.ds(o, _L)] for o in _VSLOTS]
        for o, v in zip(_VSLOTS, vs):
            q = jax.lax.shift_right_logical(v * 21475, 29)
            idx_v[row, pl.ds(o, _L)] = v * _PACK - q * (_PACK * _QUARTER - 2)

    def fire(c, buf, sem):
        for r in range(_CHUNK_ROWS):
            row = _CHUNK_ROWS * c + r
            remap_row(row)
            pltpu.async_copy(
                t2_hbm.at[idx_v.at[row, pl.ds(0, _SPLIT0)]],
                buf.at[pl.ds(r * SEQ, _SPLIT0)], sem)
            pltpu.async_copy(
                t2_hbm.at[idx_v.at[row, pl.ds(_SPLIT0, _SPLIT1)]],
                buf.at[pl.ds(r * SEQ + _SPLIT0, _SPLIT1)], sem)

    def drain(buf, sem):
        pltpu.make_async_copy(
            t2_hbm.at[pl.ds(0, _BUF_ROWS)], buf, sem).wait()

    def accum_chunk(c, buf):
        for r in range(_CHUNK_ROWS):
            base = r * SEQ

            def jbody(ji, accs, base=base):
                new = list(accs)
                o = base + ji * 8
                rows = [plsc.bitcast(buf[o + jj, pl.ds(0, _L)],
                                     jnp.bfloat16) for jj in range(8)]
                p01, p23 = rows[0] + rows[1], rows[2] + rows[3]
                p45, p67 = rows[4] + rows[5], rows[6] + rows[7]
                g = (p01 + p23) + (p45 + p67)
                w = plsc.bitcast(g, jnp.uint32)
                f_hi = plsc.bitcast(w & jnp.uint32(_HIMASK), jnp.float32)
                f_lo = plsc.bitcast(w << 16, jnp.float32)
                return (new[0] + f_hi, new[1] + f_lo)

            z = jnp.zeros((_L,), jnp.float32)
            accs = lax.fori_loop(0, SEQ // 8, jbody, (z, z))
            a0 = accs[0]
            a1 = accs[1]
            rg = c * _CHUNK_ROWS + r
            out_v[rg, pl.ds(0, _L)] = a0 + bias0
            out_v[rg, pl.ds(_L, _L)] = a1 + bias1

    fire(0, buf_a, sem_a)

    def body(i, carry):
        for b in range(2):
            cbuf, csem = (buf_a, sem_a) if b == 0 else (buf_b, sem_b)
            nbuf, nsem = (buf_b, sem_b) if b == 0 else (buf_a, sem_a)
            c = 2 * i + b

            @pl.when(c < _N_CHUNKS - 1)
            def _(c=c, nbuf=nbuf, nsem=nsem):
                fire(c + 1, nbuf, nsem)

            drain(cbuf, csem)
            accum_chunk(c, cbuf)
        return carry

    lax.fori_loop(0, _N_CHUNKS // 2, body, 0)
    pltpu.sync_copy(out_v, out_hbm.at[pl.ds(wid * _B_PER_W, _B_PER_W)])


_sc_pool = functools.partial(
    pl.kernel,
    out_type=jax.ShapeDtypeStruct((BATCH, OUT_PAD), jnp.float32),
    mesh=plsc.VectorSubcoreMesh(core_axis_name="c", subcore_axis_name="s"),
    compiler_params=pltpu.CompilerParams(use_tc_tiling_on_sc=False,
                                         needs_layout_passes=False),
    scratch_types=[
        pltpu.VMEM((_B_PER_W, SEQ), jnp.int32),
        pltpu.VMEM((_BUF_ROWS, _L), jnp.float32),
        pltpu.VMEM((_BUF_ROWS, _L), jnp.float32),
        pltpu.VMEM((_B_PER_W, OUT_PAD), jnp.float32),
        pltpu.VMEM((OUT_PAD,), jnp.float32),
        pltpu.SemaphoreType.DMA,
        pltpu.SemaphoreType.DMA,
    ],
)(_sc_pool_body)


def kernel(x, table, W1, b1, W2, b2):
    b1r = b1.reshape(1, EMB)
    b2r = b2.reshape(1, OUT)
    t4, bias = _fold_table(table, W1, W2, b1r, b2r)
    t2 = t4.reshape(_PACK * _QUARTER, _L)
    pooled = _sc_pool(x.astype(jnp.int32), t2, bias.reshape(OUT_PAD))
    return pooled[:, :OUT]
